# baseline (speedup 1.0000x reference)
import functools

import jax
import jax.numpy as jnp
from jax import lax
from jax.experimental import pallas as pl
from jax.experimental.pallas import tpu as pltpu
from jax.experimental.pallas import tpu_sc as plsc

N = 10000
E = 320000
D = 128

NC = 2
NS = 16
NW = NC * NS
EPW = E // NW
CH = 40
NCH = EPW // CH
NB = 5
NCHB = NCH // NB
NBUF = 5
RPS = 632
RLAST = N - 15 * RPS


def _make_sc_segment_sum(with_counts):
  out_type = [jax.ShapeDtypeStruct((NC, N, D), jnp.float32)]
  scratch = (
      [pltpu.VMEM((NCHB, CH), jnp.int32)] * 2
      + [pltpu.VMEM((CH, D), jnp.float32)] * NBUF
      + [pltpu.VMEM_SHARED((N, D), jnp.float32)]
      + [pltpu.SemaphoreType.DMA] * (2 * NBUF)
  )
  if with_counts:
    out_type.append(jax.ShapeDtypeStruct((NC, 1, N), jnp.float32))
    scratch += [
        pltpu.VMEM((CH,), jnp.float32),
        pltpu.VMEM_SHARED((N,), jnp.float32),
    ] + [pltpu.SemaphoreType.DMA] * NBUF

  def body(x_hbm, src_hbm, dst_hbm, z_hbm, zc_hbm, o_hbm, *rest):
    if with_counts:
      out_hbm, cnt_hbm = rest[0], rest[1]
      rest = rest[2:]
    else:
      out_hbm = rest[0]
      rest = rest[1:]
    src_v, dst_v = rest[0], rest[1]
    rows = rest[2:2 + NBUF]
    acc_sh = rest[2 + NBUF]
    gsem = rest[3 + NBUF:3 + 2 * NBUF]
    ssem = rest[3 + 2 * NBUF:3 + 3 * NBUF]
    if with_counts:
      ones_v, cnt_sh = rest[3 + 3 * NBUF], rest[4 + 3 * NBUF]
      csem = rest[5 + 3 * NBUF:5 + 4 * NBUF]

    c = lax.axis_index("c")
    s = lax.axis_index("s")
    wid = s * NC + c
    base = pl.multiple_of(s * RPS, 8)

    @pl.when(s < NS - 1)
    def _():
      pltpu.sync_copy(z_hbm, acc_sh.at[pl.ds(base, RPS)])

    @pl.when(s == NS - 1)
    def _():
      pltpu.sync_copy(z_hbm.at[pl.ds(0, RLAST)], acc_sh.at[pl.ds(base, RLAST)])

    if with_counts:
      pltpu.sync_copy(o_hbm, ones_v)

      @pl.when(s == 0)
      def _():
        pltpu.sync_copy(zc_hbm.at[0], cnt_sh)

    plsc.subcore_barrier()

    def gather(j, buf, sem):
      pltpu.async_copy(x_hbm.at[src_v.at[j]], buf, sem)

    def wait_gather(j, buf, sem):
      pltpu.make_async_copy(x_hbm.at[src_v.at[j]], buf, sem).wait()

    def scatter(j, buf, k):
      pltpu.async_copy(buf, acc_sh.at[dst_v.at[j]], ssem[k], add=True)
      if with_counts:
        pltpu.async_copy(ones_v, cnt_sh.at[dst_v.at[j]], csem[k], add=True)

    def wait_scatter(j, buf, k):
      pltpu.make_async_copy(buf, acc_sh.at[dst_v.at[j]], ssem[k]).wait()
      if with_counts:
        pltpu.make_async_copy(ones_v, cnt_sh.at[dst_v.at[j]], csem[k]).wait()

    for b in range(NB):
      pltpu.sync_copy(src_hbm.at[wid, b], src_v)
      pltpu.sync_copy(dst_hbm.at[wid, b], dst_v)

      for k in range(NBUF):
        gather(k, rows[k], gsem[k])

      def chunk_body(jj, _):
        j0 = jj * NBUF
        for k in range(NBUF):
          wait_gather(j0 + k, rows[k], gsem[k])
          scatter(j0 + k, rows[k], k)
        for k in range(NBUF):
          @pl.when(j0 + k + NBUF < NCHB)
          def _():
            wait_scatter(j0 + k, rows[k], k)
            gather(j0 + k + NBUF, rows[k], gsem[k])
        return 0
      lax.fori_loop(0, NCHB // NBUF, chunk_body, 0)

      for k in range(NBUF):
        wait_scatter(NCHB - NBUF + k, rows[k], k)

    plsc.subcore_barrier()

    @pl.when(s < NS - 1)
    def _():
      pltpu.sync_copy(acc_sh.at[pl.ds(base, RPS)], out_hbm.at[c, pl.ds(base, RPS)])

    @pl.when(s == NS - 1)
    def _():
      pltpu.sync_copy(acc_sh.at[pl.ds(base, RLAST)],
                      out_hbm.at[c, pl.ds(base, RLAST)])

    if with_counts:
      @pl.when(s == 0)
      def _():
        pltpu.sync_copy(cnt_sh, cnt_hbm.at[c, 0])

  mesh = plsc.VectorSubcoreMesh(core_axis_name="c", subcore_axis_name="s")
  return pl.kernel(body, out_type=out_type, mesh=mesh, scratch_types=scratch)


_sc_sum_counts = _make_sc_segment_sum(True)
_sc_sum = _make_sc_segment_sum(False)


BN = 400


def _pre_body(x_ref, wr_ref, b_ref, o_ref):
  o_ref[...] = jnp.dot(
      x_ref[...], wr_ref[...], preferred_element_type=jnp.float32) + b_ref[...]


def _pre(x, W_r, b_l):
  return pl.pallas_call(
      _pre_body,
      grid=(N // BN,),
      in_specs=[
          pl.BlockSpec((BN, D), lambda i: (i, 0)),
          pl.BlockSpec((D, D), lambda i: (0, 0)),
          pl.BlockSpec((1, D), lambda i: (0, 0)),
      ],
      out_specs=pl.BlockSpec((BN, D), lambda i: (i, 0)),
      out_shape=jax.ShapeDtypeStruct((N, D), jnp.float32),
      compiler_params=pltpu.CompilerParams(
          dimension_semantics=("parallel",)),
  )(x, W_r, b_l)


def _post_body(p_ref, inv_ref, pre_ref, wl_ref, o_ref):
  agg = (p_ref[0] + p_ref[1]) * inv_ref[...]
  o_ref[...] = jnp.maximum(
      jnp.dot(agg, wl_ref[...], preferred_element_type=jnp.float32)
      + pre_ref[...], 0.0)


def _post(partials, invb, pre, W_l):
  return pl.pallas_call(
      _post_body,
      grid=(N // BN,),
      in_specs=[
          pl.BlockSpec((NC, BN, D), lambda i: (0, i, 0)),
          pl.BlockSpec((BN, 1), lambda i: (i, 0)),
          pl.BlockSpec((BN, D), lambda i: (i, 0)),
          pl.BlockSpec((D, D), lambda i: (0, 0)),
      ],
      out_specs=pl.BlockSpec((BN, D), lambda i: (i, 0)),
      out_shape=jax.ShapeDtypeStruct((N, D), jnp.float32),
      compiler_params=pltpu.CompilerParams(
          dimension_semantics=("parallel",)),
  )(partials, invb, pre, W_l)


def kernel(x, edge_index, W1_l, b1_l, W1_r, W2_l, b2_l, W2_r):
  src = edge_index[0].reshape(NW, NB, NCHB, CH)
  dst = edge_index[1].reshape(NW, NB, NCHB, CH)

  z = jnp.zeros((RPS, D), jnp.float32)
  zc = jnp.zeros((1, N), jnp.float32)
  ones = jnp.ones((CH,), jnp.float32)

  pre1 = _pre(x, W1_r, b1_l.reshape(1, D))
  sums1, cnts = _sc_sum_counts(x, src, dst, z, zc, ones)
  inv = (1.0 / jnp.maximum(cnts[0, 0] + cnts[1, 0], 1.0))[:, None]

  h = _post(sums1, inv, pre1, W1_l)
  pre2 = _pre(h, W2_r, b2_l.reshape(1, D))
  (sums2,) = _sc_sum(h, src, dst, z, zc, ones)
  out = _post(sums2, inv, pre2, W2_l)
  return out


# speedup vs baseline: 1.0554x; 1.0554x over previous
import functools

import jax
import jax.numpy as jnp
from jax import lax
from jax.experimental import pallas as pl
from jax.experimental.pallas import tpu as pltpu
from jax.experimental.pallas import tpu_sc as plsc

N = 10000
E = 320000
D = 128

NC = 2
NS = 16
NW = NC * NS
EPW = E // NW
CH = 40
NCH = EPW // CH
NB = 5
NCHB = NCH // NB
NBUF = 5
RPS = 632
RLAST = N - 15 * RPS


def _make_sc_segment_sum(with_counts):
  out_type = [jax.ShapeDtypeStruct((NC, N, D), jnp.float32)]
  scratch = (
      [pltpu.VMEM((NCHB, CH), jnp.int32)] * 2
      + [pltpu.VMEM((CH, D), jnp.float32)] * NBUF
      + [pltpu.VMEM_SHARED((N, D), jnp.float32)]
      + [pltpu.SemaphoreType.DMA] * (2 * NBUF)
  )
  if with_counts:
    out_type.append(jax.ShapeDtypeStruct((NC, 1, N), jnp.float32))
    scratch += [
        pltpu.VMEM((CH,), jnp.float32),
        pltpu.VMEM_SHARED((N,), jnp.float32),
    ] + [pltpu.SemaphoreType.DMA] * NBUF

  def body(x_hbm, src_hbm, dst_hbm, z_hbm, zc_hbm, o_hbm, *rest):
    if with_counts:
      out_hbm, cnt_hbm = rest[0], rest[1]
      rest = rest[2:]
    else:
      out_hbm = rest[0]
      rest = rest[1:]
    src_v, dst_v = rest[0], rest[1]
    rows = rest[2:2 + NBUF]
    acc_sh = rest[2 + NBUF]
    gsem = rest[3 + NBUF:3 + 2 * NBUF]
    ssem = rest[3 + 2 * NBUF:3 + 3 * NBUF]
    if with_counts:
      ones_v, cnt_sh = rest[3 + 3 * NBUF], rest[4 + 3 * NBUF]
      csem = rest[5 + 3 * NBUF:5 + 4 * NBUF]

    c = lax.axis_index("c")
    s = lax.axis_index("s")
    wid = s * NC + c
    base = pl.multiple_of(s * RPS, 8)

    @pl.when(s < NS - 1)
    def _():
      pltpu.sync_copy(z_hbm, acc_sh.at[pl.ds(base, RPS)])

    @pl.when(s == NS - 1)
    def _():
      pltpu.sync_copy(z_hbm.at[pl.ds(0, RLAST)], acc_sh.at[pl.ds(base, RLAST)])

    if with_counts:
      pltpu.sync_copy(o_hbm, ones_v)

      @pl.when(s == 0)
      def _():
        pltpu.sync_copy(zc_hbm.at[0], cnt_sh)

    plsc.subcore_barrier()

    def gather(j, buf, sem):
      pltpu.async_copy(x_hbm.at[src_v.at[j]], buf, sem)

    def wait_gather(j, buf, sem):
      pltpu.make_async_copy(x_hbm.at[src_v.at[j]], buf, sem).wait()

    def scatter(j, buf, k):
      pltpu.async_copy(buf, acc_sh.at[dst_v.at[j]], ssem[k], add=True)
      if with_counts:
        pltpu.async_copy(ones_v, cnt_sh.at[dst_v.at[j]], csem[k], add=True)

    def wait_scatter(j, buf, k):
      pltpu.make_async_copy(buf, acc_sh.at[dst_v.at[j]], ssem[k]).wait()
      if with_counts:
        pltpu.make_async_copy(ones_v, cnt_sh.at[dst_v.at[j]], csem[k]).wait()

    for b in range(NB):
      pltpu.sync_copy(src_hbm.at[wid, b], src_v)
      pltpu.sync_copy(dst_hbm.at[wid, b], dst_v)

      for k in range(NBUF):
        gather(k, rows[k], gsem[k])

      def chunk_body(jj, _):
        j0 = jj * NBUF
        for k in range(NBUF):
          wait_gather(j0 + k, rows[k], gsem[k])
          scatter(j0 + k, rows[k], k)
        for k in range(NBUF):
          @pl.when(j0 + k + NBUF < NCHB)
          def _():
            wait_scatter(j0 + k, rows[k], k)
            gather(j0 + k + NBUF, rows[k], gsem[k])
        return 0
      lax.fori_loop(0, NCHB // NBUF, chunk_body, 0)

      for k in range(NBUF):
        wait_scatter(NCHB - NBUF + k, rows[k], k)

    plsc.subcore_barrier()

    @pl.when(s < NS - 1)
    def _():
      pltpu.sync_copy(acc_sh.at[pl.ds(base, RPS)], out_hbm.at[c, pl.ds(base, RPS)])

    @pl.when(s == NS - 1)
    def _():
      pltpu.sync_copy(acc_sh.at[pl.ds(base, RLAST)],
                      out_hbm.at[c, pl.ds(base, RLAST)])

    if with_counts:
      @pl.when(s == 0)
      def _():
        pltpu.sync_copy(cnt_sh, cnt_hbm.at[c, 0])

  mesh = plsc.VectorSubcoreMesh(core_axis_name="c", subcore_axis_name="s")
  return pl.kernel(body, out_type=out_type, mesh=mesh, scratch_types=scratch)


_sc_sum_counts = _make_sc_segment_sum(True)
_sc_sum = _make_sc_segment_sum(False)


BN = 1000


def _dense_body(p_ref, inv_ref, x_ref, wl_ref, wr_ref, b_ref, o_ref):
  agg = (p_ref[0] + p_ref[1]) * inv_ref[...]
  o_ref[...] = jnp.maximum(
      jnp.dot(agg, wl_ref[...], preferred_element_type=jnp.float32)
      + jnp.dot(x_ref[...], wr_ref[...], preferred_element_type=jnp.float32)
      + b_ref[...], 0.0)


def _dense(partials, invb, x, W_l, W_r, b_l):
  return pl.pallas_call(
      _dense_body,
      grid=(N // BN,),
      in_specs=[
          pl.BlockSpec((NC, BN, D), lambda i: (0, i, 0)),
          pl.BlockSpec((BN, 1), lambda i: (i, 0)),
          pl.BlockSpec((BN, D), lambda i: (i, 0)),
          pl.BlockSpec((D, D), lambda i: (0, 0)),
          pl.BlockSpec((D, D), lambda i: (0, 0)),
          pl.BlockSpec((1, D), lambda i: (0, 0)),
      ],
      out_specs=pl.BlockSpec((BN, D), lambda i: (i, 0)),
      out_shape=jax.ShapeDtypeStruct((N, D), jnp.float32),
      compiler_params=pltpu.CompilerParams(
          dimension_semantics=("parallel",)),
  )(partials, invb, x, W_l, W_r, b_l)


def kernel(x, edge_index, W1_l, b1_l, W1_r, W2_l, b2_l, W2_r):
  src = edge_index[0].reshape(NW, NB, NCHB, CH)
  dst = edge_index[1].reshape(NW, NB, NCHB, CH)

  z = jnp.zeros((RPS, D), jnp.float32)
  zc = jnp.zeros((1, N), jnp.float32)
  ones = jnp.ones((CH,), jnp.float32)

  sums1, cnts = _sc_sum_counts(x, src, dst, z, zc, ones)
  inv = (1.0 / jnp.maximum(cnts[0, 0] + cnts[1, 0], 1.0))[:, None]

  h = _dense(sums1, inv, x, W1_l, W1_r, b1_l.reshape(1, D))
  (sums2,) = _sc_sum(h, src, dst, z, zc, ones)
  out = _dense(sums2, inv, h, W2_l, W2_r, b2_l.reshape(1, D))
  return out


# speedup vs baseline: 1.0734x; 1.0171x over previous
import functools

import jax
import jax.numpy as jnp
from jax import lax
from jax.experimental import pallas as pl
from jax.experimental.pallas import tpu as pltpu
from jax.experimental.pallas import tpu_sc as plsc

N = 10000
E = 320000
D = 128

NC = 2
NS = 16
NW = NC * NS
EPW = E // NW
CH = 40
NCH = EPW // CH
NB = 5
NCHB = NCH // NB
NBUF = 5
RPS = 632
RLAST = N - 15 * RPS


def _make_sc_segment_sum(with_counts):
  out_type = [jax.ShapeDtypeStruct((NC, N, D), jnp.float32)]
  scratch = (
      [pltpu.VMEM((NCHB, CH), jnp.int32)] * 2
      + [pltpu.VMEM((CH, D), jnp.float32)] * NBUF
      + [pltpu.VMEM_SHARED((N, D), jnp.float32)]
      + [pltpu.SemaphoreType.DMA] * (2 * NBUF)
  )
  if with_counts:
    out_type.append(jax.ShapeDtypeStruct((NC, 1, N), jnp.float32))
    scratch += [
        pltpu.VMEM((CH,), jnp.float32),
        pltpu.VMEM_SHARED((N,), jnp.float32),
    ] + [pltpu.SemaphoreType.DMA] * NBUF

  def body(x_hbm, src_hbm, dst_hbm, z_hbm, zc_hbm, o_hbm, *rest):
    if with_counts:
      out_hbm, cnt_hbm = rest[0], rest[1]
      rest = rest[2:]
    else:
      out_hbm = rest[0]
      rest = rest[1:]
    src_v, dst_v = rest[0], rest[1]
    rows = rest[2:2 + NBUF]
    acc_sh = rest[2 + NBUF]
    gsem = rest[3 + NBUF:3 + 2 * NBUF]
    ssem = rest[3 + 2 * NBUF:3 + 3 * NBUF]
    if with_counts:
      ones_v, cnt_sh = rest[3 + 3 * NBUF], rest[4 + 3 * NBUF]
      csem = rest[5 + 3 * NBUF:5 + 4 * NBUF]

    c = lax.axis_index("c")
    s = lax.axis_index("s")
    wid = s * NC + c
    base = pl.multiple_of(s * RPS, 8)

    @pl.when(s < NS - 1)
    def _():
      pltpu.sync_copy(z_hbm, acc_sh.at[pl.ds(base, RPS)])

    @pl.when(s == NS - 1)
    def _():
      pltpu.sync_copy(z_hbm.at[pl.ds(0, RLAST)], acc_sh.at[pl.ds(base, RLAST)])

    if with_counts:
      pltpu.sync_copy(o_hbm, ones_v)

      @pl.when(s == 0)
      def _():
        pltpu.sync_copy(zc_hbm.at[0], cnt_sh)

    plsc.subcore_barrier()

    def gather(j, buf, sem):
      pltpu.async_copy(x_hbm.at[src_v.at[j]], buf, sem)

    def wait_gather(j, buf, sem):
      pltpu.make_async_copy(x_hbm.at[src_v.at[j]], buf, sem).wait()

    def scatter(j, buf, k):
      pltpu.async_copy(buf, acc_sh.at[dst_v.at[j]], ssem[k], add=True)
      if with_counts:
        pltpu.async_copy(ones_v, cnt_sh.at[dst_v.at[j]], csem[k], add=True)

    def wait_scatter(j, buf, k):
      pltpu.make_async_copy(buf, acc_sh.at[dst_v.at[j]], ssem[k]).wait()
      if with_counts:
        pltpu.make_async_copy(ones_v, cnt_sh.at[dst_v.at[j]], csem[k]).wait()

    for b in range(NB):
      pltpu.sync_copy(src_hbm.at[wid, b], src_v)
      pltpu.sync_copy(dst_hbm.at[wid, b], dst_v)

      for k in range(NBUF):
        gather(k, rows[k], gsem[k])

      def chunk_body(jj, _):
        j0 = jj * NBUF
        for k in range(NBUF):
          wait_gather(j0 + k, rows[k], gsem[k])
          scatter(j0 + k, rows[k], k)
        for k in range(NBUF):
          @pl.when(j0 + k + NBUF < NCHB)
          def _():
            wait_scatter(j0 + k, rows[k], k)
            gather(j0 + k + NBUF, rows[k], gsem[k])
        return 0
      lax.fori_loop(0, NCHB // NBUF, chunk_body, 0)

      for k in range(NBUF):
        wait_scatter(NCHB - NBUF + k, rows[k], k)

    plsc.subcore_barrier()

    @pl.when(s < NS - 1)
    def _():
      pltpu.sync_copy(acc_sh.at[pl.ds(base, RPS)], out_hbm.at[c, pl.ds(base, RPS)])

    @pl.when(s == NS - 1)
    def _():
      pltpu.sync_copy(acc_sh.at[pl.ds(base, RLAST)],
                      out_hbm.at[c, pl.ds(base, RLAST)])

    if with_counts:
      @pl.when(s == 0)
      def _():
        pltpu.sync_copy(cnt_sh, cnt_hbm.at[c, 0])

  mesh = plsc.VectorSubcoreMesh(core_axis_name="c", subcore_axis_name="s")
  return pl.kernel(body, out_type=out_type, mesh=mesh, scratch_types=scratch)


_sc_sum_counts = _make_sc_segment_sum(True)
_sc_sum = _make_sc_segment_sum(False)


BN = 2000


def _dense_body(p_ref, inv_ref, x_ref, wl_ref, wr_ref, b_ref, o_ref):
  agg = (p_ref[0] + p_ref[1]) * inv_ref[...]
  o_ref[...] = jnp.maximum(
      jnp.dot(agg, wl_ref[...], preferred_element_type=jnp.float32)
      + jnp.dot(x_ref[...], wr_ref[...], preferred_element_type=jnp.float32)
      + b_ref[...], 0.0)


def _dense(partials, invb, x, W_l, W_r, b_l):
  return pl.pallas_call(
      _dense_body,
      grid=(N // BN,),
      in_specs=[
          pl.BlockSpec((NC, BN, D), lambda i: (0, i, 0)),
          pl.BlockSpec((BN, 1), lambda i: (i, 0)),
          pl.BlockSpec((BN, D), lambda i: (i, 0)),
          pl.BlockSpec((D, D), lambda i: (0, 0)),
          pl.BlockSpec((D, D), lambda i: (0, 0)),
          pl.BlockSpec((1, D), lambda i: (0, 0)),
      ],
      out_specs=pl.BlockSpec((BN, D), lambda i: (i, 0)),
      out_shape=jax.ShapeDtypeStruct((N, D), jnp.float32),
      compiler_params=pltpu.CompilerParams(
          dimension_semantics=("parallel",)),
  )(partials, invb, x, W_l, W_r, b_l)


def kernel(x, edge_index, W1_l, b1_l, W1_r, W2_l, b2_l, W2_r):
  src = edge_index[0].reshape(NW, NB, NCHB, CH)
  dst = edge_index[1].reshape(NW, NB, NCHB, CH)

  z = jnp.zeros((RPS, D), jnp.float32)
  zc = jnp.zeros((1, N), jnp.float32)
  ones = jnp.ones((CH,), jnp.float32)

  sums1, cnts = _sc_sum_counts(x, src, dst, z, zc, ones)
  inv = (1.0 / jnp.maximum(cnts[0, 0] + cnts[1, 0], 1.0))[:, None]

  h = _dense(sums1, inv, x, W1_l, W1_r, b1_l.reshape(1, D))
  (sums2,) = _sc_sum(h, src, dst, z, zc, ones)
  out = _dense(sums2, inv, h, W2_l, W2_r, b2_l.reshape(1, D))
  return out
